# baseline (speedup 1.0000x reference)
import jax
import jax.numpy as jnp
from jax.experimental import pallas as pl
from jax.experimental.pallas import tpu as pltpu

_NEG = -9000000000000000.0
_BI = 128


def _gat_step(x_ref, pos_ref, adj_ref, w_ref, a1_ref, a2_ref, wpt_ref, bp_ref,
              out_ref, wh_ref, f2_ref):
    i = pl.program_id(1)
    bi = out_ref.shape[1]

    @pl.when(i == 0)
    def _compute_wh():
        wh_all = jnp.dot(x_ref[0], w_ref[...], preferred_element_type=jnp.float32)
        wh_ref[...] = wh_all
        f2_ref[...] = jax.lax.dot_general(
            a2_ref[...], wh_all, (((1,), (1,)), ((), ())),
            preferred_element_type=jnp.float32)

    wh = wh_ref[...]
    wh_i = wh_ref[pl.ds(i * bi, bi), :]
    f1 = jnp.dot(wh_i, a1_ref[...], preferred_element_type=jnp.float32)
    e = f1 + f2_ref[...]
    e = jnp.where(e >= 0.0, e, 0.2 * e)
    e = jnp.where(adj_ref[0] > 0.0, e, _NEG)
    m = jnp.max(e, axis=1, keepdims=True)
    p = jnp.exp(e - m)
    att = p / jnp.sum(p, axis=1, keepdims=True)
    h = jnp.dot(att, wh, preferred_element_type=jnp.float32)
    pe = jnp.dot(pos_ref[0], wpt_ref[...], preferred_element_type=jnp.float32)
    pe = jnp.maximum(pe + bp_ref[...], 0.0)
    h = h + pe
    out_ref[0] = jnp.where(h > 0.0, h, jnp.exp(h) - 1.0)


def kernel(x, pos, adj, W, a, W_pos, b_pos):
    B, N, F_in = x.shape
    F_out = W.shape[1]
    a1 = a[:F_out]
    a2 = a[F_out:, 0].reshape(1, F_out)
    wpt = W_pos.T
    bp = b_pos.reshape(1, F_out)

    grid = (B, N // _BI)
    return pl.pallas_call(
        _gat_step,
        grid=grid,
        in_specs=[
            pl.BlockSpec((1, N, F_in), lambda b, i: (b, 0, 0)),
            pl.BlockSpec((1, _BI, 3), lambda b, i: (b, i, 0)),
            pl.BlockSpec((1, _BI, N), lambda b, i: (b, i, 0)),
            pl.BlockSpec((F_in, F_out), lambda b, i: (0, 0)),
            pl.BlockSpec((F_out, 1), lambda b, i: (0, 0)),
            pl.BlockSpec((1, F_out), lambda b, i: (0, 0)),
            pl.BlockSpec((3, F_out), lambda b, i: (0, 0)),
            pl.BlockSpec((1, F_out), lambda b, i: (0, 0)),
        ],
        out_specs=pl.BlockSpec((1, _BI, F_out), lambda b, i: (b, i, 0)),
        out_shape=jax.ShapeDtypeStruct((B, N, F_out), jnp.float32),
        scratch_shapes=[
            pltpu.VMEM((N, F_out), jnp.float32),
            pltpu.VMEM((1, N), jnp.float32),
        ],
    )(x, pos, adj, W, a1, a2, wpt, bp)


# speedup vs baseline: 1.5338x; 1.5338x over previous
import jax
import jax.numpy as jnp
from jax.experimental import pallas as pl

_NEG = -9000000000000000.0


def _gat_step(x_ref, pos_ref, adj_ref, w_ref, a1_ref, a2_ref, wpt_ref, bp_ref,
              out_ref):
    wh = jnp.dot(x_ref[0], w_ref[...], preferred_element_type=jnp.float32)
    f1 = jnp.dot(wh, a1_ref[...], preferred_element_type=jnp.float32)
    f2 = jax.lax.dot_general(
        a2_ref[...], wh, (((1,), (1,)), ((), ())),
        preferred_element_type=jnp.float32)
    e = f1 + f2
    e = jnp.maximum(e, 0.2 * e)
    e = jnp.where(adj_ref[0] > 0.0, e, _NEG)
    m = jnp.max(e, axis=1, keepdims=True)
    p = jnp.exp(e - m)
    att = p / jnp.sum(p, axis=1, keepdims=True)
    h = jnp.dot(att, wh, preferred_element_type=jnp.float32)
    pe = jnp.dot(pos_ref[0], wpt_ref[...], preferred_element_type=jnp.float32)
    pe = jnp.maximum(pe + bp_ref[...], 0.0)
    h = h + pe
    out_ref[0] = jnp.where(h > 0.0, h, jnp.exp(h) - 1.0)


def kernel(x, pos, adj, W, a, W_pos, b_pos):
    B, N, F_in = x.shape
    F_out = W.shape[1]
    a1 = a[:F_out]
    a2 = a[F_out:, 0].reshape(1, F_out)
    wpt = W_pos.T
    bp = b_pos.reshape(1, F_out)

    return pl.pallas_call(
        _gat_step,
        grid=(B,),
        in_specs=[
            pl.BlockSpec((1, N, F_in), lambda b: (b, 0, 0)),
            pl.BlockSpec((1, N, 3), lambda b: (b, 0, 0)),
            pl.BlockSpec((1, N, N), lambda b: (b, 0, 0)),
            pl.BlockSpec((F_in, F_out), lambda b: (0, 0)),
            pl.BlockSpec((F_out, 1), lambda b: (0, 0)),
            pl.BlockSpec((1, F_out), lambda b: (0, 0)),
            pl.BlockSpec((3, F_out), lambda b: (0, 0)),
            pl.BlockSpec((1, F_out), lambda b: (0, 0)),
        ],
        out_specs=pl.BlockSpec((1, N, F_out), lambda b: (b, 0, 0)),
        out_shape=jax.ShapeDtypeStruct((B, N, F_out), jnp.float32),
    )(x, pos, adj, W, a1, a2, wpt, bp)
